# Initial kernel scaffold; baseline (speedup 1.0000x reference)
import jax
import jax.numpy as jnp
from jax.experimental import pallas as pl


def kernel(pred, target):
    raise NotImplementedError("write your pallas kernel here")


# speedup vs baseline: 2.7318x; 2.7318x over previous
import functools
import math

import jax
import jax.numpy as jnp
from jax import lax
from jax.experimental import pallas as pl
from jax.experimental.pallas import tpu as pltpu
from jax.experimental.pallas import tpu_sc as plsc

_N = 4096
_V = 32000
_PAD = 0
_SMOOTH = 0.1
_CONF = 1.0 - _SMOOTH
_EPS = _SMOOTH / (_V - 2)
_C = _CONF * math.log(_CONF) + _SMOOTH * math.log(_EPS)
_COEF_T = _CONF - _EPS

_ROWS_PER_STEP = 32
_NUM_STEPS = _N // _ROWS_PER_STEP

_NUM_WORKERS = 32
_ROWS_PER_WORKER = _N // _NUM_WORKERS
_LANES = 16
_SLICES = _ROWS_PER_WORKER // _LANES


def _sc_body(pred_hbm, tgt_hbm, out_hbm, tgt_v, idxt_v, idx0_v, gt_v, g0_v,
             acc_v, sem):
    wid = lax.axis_index("s") * 2 + lax.axis_index("c")
    base = wid * _ROWS_PER_WORKER
    pltpu.sync_copy(tgt_hbm.at[pl.ds(base, _ROWS_PER_WORKER)], tgt_v)
    for j in range(_SLICES):
        t = tgt_v[pl.ds(j * _LANES, _LANES)]
        rows = (base + j * _LANES) + lax.iota(jnp.int32, _LANES)
        idx0_v[pl.ds(j * _LANES, _LANES)] = rows * _V
        idxt_v[pl.ds(j * _LANES, _LANES)] = rows * _V + t
    pltpu.async_copy(pred_hbm.at[idxt_v], gt_v, sem).wait()
    pltpu.async_copy(pred_hbm.at[idx0_v], g0_v, sem).wait()
    acc = jnp.zeros((_LANES,), jnp.float32)
    for j in range(_SLICES):
        t = tgt_v[pl.ds(j * _LANES, _LANES)]
        gv = gt_v[pl.ds(j * _LANES, _LANES)]
        g0 = g0_v[pl.ds(j * _LANES, _LANES)]
        acc = acc + jnp.where(t != _PAD, _EPS * g0 - _COEF_T * gv, 0.0)
    acc_v[...] = acc
    pltpu.sync_copy(acc_v, out_hbm.at[wid])


@functools.lru_cache(maxsize=None)
def _sc_gather_fn():
    return pl.kernel(
        _sc_body,
        mesh=plsc.VectorSubcoreMesh(core_axis_name="c", subcore_axis_name="s"),
        out_type=jax.ShapeDtypeStruct((_NUM_WORKERS, _LANES), jnp.float32),
        scratch_types=[
            pltpu.VMEM((_ROWS_PER_WORKER,), jnp.int32),
            pltpu.VMEM((_ROWS_PER_WORKER,), jnp.int32),
            pltpu.VMEM((_ROWS_PER_WORKER,), jnp.int32),
            pltpu.VMEM((_ROWS_PER_WORKER,), jnp.float32),
            pltpu.VMEM((_ROWS_PER_WORKER,), jnp.float32),
            pltpu.VMEM((_LANES,), jnp.float32),
            pltpu.SemaphoreType.DMA,
        ],
    )


def _tc_body(tgt_ref, sc_ref, x_ref, out_ref):
    i = pl.program_id(0)
    x = x_ref[...]
    m = jnp.max(x, axis=1, keepdims=True)
    se = jnp.sum(jnp.exp(x - m), axis=1, keepdims=True)
    lse = m + jnp.log(se)
    p = jnp.sum(x, axis=1, keepdims=True)
    mask = (tgt_ref[...] != _PAD).astype(jnp.float32)
    part = jnp.sum(mask * (_C + lse - _EPS * p))
    prev = jnp.where(i == 0, jnp.sum(sc_ref[...]), out_ref[0, 0])
    tot = prev + part
    out_ref[0, 0] = jnp.where(i == _NUM_STEPS - 1, tot * (1.0 / _N), tot)


def _tc_loss(pred, tgt2d, sc_parts):
    return pl.pallas_call(
        _tc_body,
        grid=(_NUM_STEPS,),
        in_specs=[
            pl.BlockSpec((_ROWS_PER_STEP, 1), lambda i: (i, 0)),
            pl.BlockSpec((_NUM_WORKERS, _LANES), lambda i: (0, 0)),
            pl.BlockSpec((_ROWS_PER_STEP, _V), lambda i: (i, 0)),
        ],
        out_specs=pl.BlockSpec(memory_space=pltpu.SMEM),
        out_shape=jax.ShapeDtypeStruct((1, 1), jnp.float32),
        compiler_params=pltpu.CompilerParams(
            dimension_semantics=("arbitrary",)),
    )(tgt2d, sc_parts, pred)


def kernel(pred, target):
    sc_parts = _sc_gather_fn()(pred.reshape(-1), target)
    loss = _tc_loss(pred, target.reshape(_N, 1), sc_parts)
    return loss[0, 0]


# speedup vs baseline: 2.9684x; 1.0866x over previous
import functools
import math

import jax
import jax.numpy as jnp
from jax import lax
from jax.experimental import pallas as pl
from jax.experimental.pallas import tpu as pltpu
from jax.experimental.pallas import tpu_sc as plsc

_N = 4096
_V = 32000
_PAD = 0
_SMOOTH = 0.1
_CONF = 1.0 - _SMOOTH
_EPS = _SMOOTH / (_V - 2)
_C = _CONF * math.log(_CONF) + _SMOOTH * math.log(_EPS)
_COEF_T = _CONF - _EPS

_ROWS_PER_STEP = 128
_NUM_STEPS = _N // _ROWS_PER_STEP

_NUM_WORKERS = 32
_ROWS_PER_WORKER = _N // _NUM_WORKERS
_LANES = 16
_SLICES = _ROWS_PER_WORKER // _LANES


def _sc_body(pred_hbm, tgt_hbm, out_hbm, tgt_v, idxt_v, idx0_v, gt_v, g0_v,
             acc_v, sem):
    wid = lax.axis_index("s") * 2 + lax.axis_index("c")
    base = wid * _ROWS_PER_WORKER
    pltpu.sync_copy(tgt_hbm.at[pl.ds(base, _ROWS_PER_WORKER)], tgt_v)
    for j in range(_SLICES):
        t = tgt_v[pl.ds(j * _LANES, _LANES)]
        rows = (base + j * _LANES) + lax.iota(jnp.int32, _LANES)
        idx0_v[pl.ds(j * _LANES, _LANES)] = rows * _V
        idxt_v[pl.ds(j * _LANES, _LANES)] = rows * _V + t
    pltpu.async_copy(pred_hbm.at[idxt_v], gt_v, sem).wait()
    pltpu.async_copy(pred_hbm.at[idx0_v], g0_v, sem).wait()
    acc = jnp.zeros((_LANES,), jnp.float32)
    for j in range(_SLICES):
        t = tgt_v[pl.ds(j * _LANES, _LANES)]
        gv = gt_v[pl.ds(j * _LANES, _LANES)]
        g0 = g0_v[pl.ds(j * _LANES, _LANES)]
        acc = acc + jnp.where(t != _PAD, _EPS * g0 - _COEF_T * gv, 0.0)
    acc_v[...] = acc
    pltpu.sync_copy(acc_v, out_hbm.at[wid])


@functools.lru_cache(maxsize=None)
def _sc_gather_fn():
    return pl.kernel(
        _sc_body,
        mesh=plsc.VectorSubcoreMesh(core_axis_name="c", subcore_axis_name="s"),
        out_type=jax.ShapeDtypeStruct((_NUM_WORKERS, _LANES), jnp.float32),
        scratch_types=[
            pltpu.VMEM((_ROWS_PER_WORKER,), jnp.int32),
            pltpu.VMEM((_ROWS_PER_WORKER,), jnp.int32),
            pltpu.VMEM((_ROWS_PER_WORKER,), jnp.int32),
            pltpu.VMEM((_ROWS_PER_WORKER,), jnp.float32),
            pltpu.VMEM((_ROWS_PER_WORKER,), jnp.float32),
            pltpu.VMEM((_LANES,), jnp.float32),
            pltpu.SemaphoreType.DMA,
        ],
    )


def _tc_body(tgt_ref, sc_ref, x_ref, out_ref):
    i = pl.program_id(0)
    x = x_ref[...]
    m = jnp.max(x, axis=1, keepdims=True)
    se = jnp.sum(jnp.exp(x - m), axis=1, keepdims=True)
    lse = m + jnp.log(se)
    p = jnp.sum(x, axis=1, keepdims=True)
    mask = (tgt_ref[...] != _PAD).astype(jnp.float32)
    part = jnp.sum(mask * (_C + lse - _EPS * p))
    prev = jnp.where(i == 0, jnp.sum(sc_ref[...]), out_ref[0, 0])
    tot = prev + part
    out_ref[0, 0] = jnp.where(i == _NUM_STEPS - 1, tot * (1.0 / _N), tot)


def _tc_loss(pred, tgt2d, sc_parts):
    return pl.pallas_call(
        _tc_body,
        grid=(_NUM_STEPS,),
        in_specs=[
            pl.BlockSpec((_ROWS_PER_STEP, 1), lambda i: (i, 0)),
            pl.BlockSpec((_NUM_WORKERS, _LANES), lambda i: (0, 0)),
            pl.BlockSpec((_ROWS_PER_STEP, _V), lambda i: (i, 0)),
        ],
        out_specs=pl.BlockSpec(memory_space=pltpu.SMEM),
        out_shape=jax.ShapeDtypeStruct((1, 1), jnp.float32),
        compiler_params=pltpu.CompilerParams(
            dimension_semantics=("arbitrary",)),
    )(tgt2d, sc_parts, pred)


def kernel(pred, target):
    sc_parts = _sc_gather_fn()(pred.reshape(-1), target)
    loss = _tc_loss(pred, target.reshape(_N, 1), sc_parts)
    return loss[0, 0]


# speedup vs baseline: 3.0143x; 1.0155x over previous
import functools
import math

import jax
import jax.numpy as jnp
from jax import lax
from jax.experimental import pallas as pl
from jax.experimental.pallas import tpu as pltpu
from jax.experimental.pallas import tpu_sc as plsc

_N = 4096
_V = 32000
_PAD = 0
_SMOOTH = 0.1
_CONF = 1.0 - _SMOOTH
_EPS = _SMOOTH / (_V - 2)
_C = _CONF * math.log(_CONF) + _SMOOTH * math.log(_EPS)
_COEF_T = _CONF - _EPS

_LANES = 16
_NUM_WORKERS = 32

_NSC = 1024
_NTC = _N - _NSC

_ROWS_PER_STEP = 128
_NUM_STEPS = _NTC // _ROWS_PER_STEP

_ROWS_PER_WORKER = _N // _NUM_WORKERS
_SLICES = _ROWS_PER_WORKER // _LANES

_RPT = _NSC // _NUM_WORKERS
_VREGS = _V // _LANES
_UNROLL = 16
_INNER = _VREGS // _UNROLL


def _sc_gather_body(pred_hbm, tgt_hbm, out_hbm, tgt_v, idxt_v, idx0_v, gt_v,
                    g0_v, acc_v, sem):
    wid = lax.axis_index("s") * 2 + lax.axis_index("c")
    base = wid * _ROWS_PER_WORKER
    pltpu.sync_copy(tgt_hbm.at[pl.ds(base, _ROWS_PER_WORKER)], tgt_v)
    for j in range(_SLICES):
        t = tgt_v[pl.ds(j * _LANES, _LANES)]
        rows = (base + j * _LANES) + lax.iota(jnp.int32, _LANES)
        idx0_v[pl.ds(j * _LANES, _LANES)] = rows * _V
        idxt_v[pl.ds(j * _LANES, _LANES)] = rows * _V + t
    pltpu.async_copy(pred_hbm.at[idxt_v], gt_v, sem).wait()
    pltpu.async_copy(pred_hbm.at[idx0_v], g0_v, sem).wait()
    acc = jnp.zeros((_LANES,), jnp.float32)
    for j in range(_SLICES):
        t = tgt_v[pl.ds(j * _LANES, _LANES)]
        gv = gt_v[pl.ds(j * _LANES, _LANES)]
        g0 = g0_v[pl.ds(j * _LANES, _LANES)]
        acc = acc + jnp.where(t != _PAD, _EPS * g0 - _COEF_T * gv, 0.0)
    acc_v[...] = acc
    pltpu.sync_copy(acc_v, out_hbm.at[wid])


@functools.lru_cache(maxsize=None)
def _sc_gather_fn():
    return pl.kernel(
        _sc_gather_body,
        mesh=plsc.VectorSubcoreMesh(core_axis_name="c", subcore_axis_name="s"),
        out_type=jax.ShapeDtypeStruct((_NUM_WORKERS, _LANES), jnp.float32),
        scratch_types=[
            pltpu.VMEM((_ROWS_PER_WORKER,), jnp.int32),
            pltpu.VMEM((_ROWS_PER_WORKER,), jnp.int32),
            pltpu.VMEM((_ROWS_PER_WORKER,), jnp.int32),
            pltpu.VMEM((_ROWS_PER_WORKER,), jnp.float32),
            pltpu.VMEM((_ROWS_PER_WORKER,), jnp.float32),
            pltpu.VMEM((_LANES,), jnp.float32),
            pltpu.SemaphoreType.DMA,
        ],
    )


def _sc_row_stats(buf, local, outm_v, outs_v, outp_v):
    def pass1(i, carry):
        m_acc, p_acc = carry
        for u in range(_UNROLL):
            x = buf[pl.ds((i * _UNROLL + u) * _LANES, _LANES)]
            m_acc = jnp.maximum(m_acc, x)
            p_acc = p_acc + x
        return m_acc, p_acc

    m_acc, p_acc = lax.fori_loop(
        0, _INNER, pass1,
        (jnp.full((_LANES,), -jnp.inf, jnp.float32),
         jnp.zeros((_LANES,), jnp.float32)))

    def pass2(i, s_acc):
        for u in range(_UNROLL):
            x = buf[pl.ds((i * _UNROLL + u) * _LANES, _LANES)]
            s_acc = s_acc + jnp.exp(x - m_acc)
        return s_acc

    s_acc = lax.fori_loop(0, _INNER, pass2, jnp.zeros((_LANES,), jnp.float32))
    outm_v[pl.ds(local * _LANES, _LANES)] = m_acc
    outs_v[pl.ds(local * _LANES, _LANES)] = s_acc
    outp_v[pl.ds(local * _LANES, _LANES)] = p_acc


def _sc_dense_body(pred_hbm, outm_hbm, outs_hbm, outp_hbm, buf0, buf1,
                   outm_v, outs_v, outp_v, sem0, sem1):
    wid = lax.axis_index("s") * 2 + lax.axis_index("c")
    row0 = _NTC + wid * _RPT

    pltpu.async_copy(pred_hbm.at[pl.ds(row0 * _V, _V)], buf0, sem0).wait()

    def pair(kk, _):
        r = row0 + 2 * kk
        nxt = jnp.minimum(r + 1, _N - 1)
        cp1 = pltpu.async_copy(pred_hbm.at[pl.ds(nxt * _V, _V)], buf1, sem1)
        _sc_row_stats(buf0, 2 * kk, outm_v, outs_v, outp_v)
        cp1.wait()
        nxt2 = jnp.minimum(r + 2, _N - 1)
        cp0 = pltpu.async_copy(pred_hbm.at[pl.ds(nxt2 * _V, _V)], buf0, sem0)
        _sc_row_stats(buf1, 2 * kk + 1, outm_v, outs_v, outp_v)
        cp0.wait()
        return 0

    lax.fori_loop(0, _RPT // 2, pair, 0)

    base = wid * _RPT * _LANES
    pltpu.sync_copy(outm_v, outm_hbm.at[pl.ds(base, _RPT * _LANES)])
    pltpu.sync_copy(outs_v, outs_hbm.at[pl.ds(base, _RPT * _LANES)])
    pltpu.sync_copy(outp_v, outp_hbm.at[pl.ds(base, _RPT * _LANES)])


@functools.lru_cache(maxsize=None)
def _sc_dense_fn():
    stats = jax.ShapeDtypeStruct((_NSC * _LANES,), jnp.float32)
    return pl.kernel(
        _sc_dense_body,
        mesh=plsc.VectorSubcoreMesh(core_axis_name="c", subcore_axis_name="s"),
        out_type=(stats, stats, stats),
        scratch_types=[
            pltpu.VMEM((_V,), jnp.float32),
            pltpu.VMEM((_V,), jnp.float32),
            pltpu.VMEM((_RPT * _LANES,), jnp.float32),
            pltpu.VMEM((_RPT * _LANES,), jnp.float32),
            pltpu.VMEM((_RPT * _LANES,), jnp.float32),
            pltpu.SemaphoreType.DMA,
            pltpu.SemaphoreType.DMA,
        ],
    )


def _tc_main_body(tgt_ref, x_ref, out_ref):
    i = pl.program_id(0)
    x = x_ref[...]
    m = jnp.max(x, axis=1, keepdims=True)
    se = jnp.sum(jnp.exp(x - m), axis=1, keepdims=True)
    lse = m + jnp.log(se)
    p = jnp.sum(x, axis=1, keepdims=True)
    mask = (tgt_ref[...] != _PAD).astype(jnp.float32)
    part = jnp.sum(mask * (_C + lse - _EPS * p))
    prev = jnp.where(i == 0, 0.0, out_ref[0, 0])
    out_ref[0, 0] = prev + part


def _tc_main(pred, tgt2d):
    return pl.pallas_call(
        _tc_main_body,
        grid=(_NUM_STEPS,),
        in_specs=[
            pl.BlockSpec((_ROWS_PER_STEP, 1), lambda i: (i, 0)),
            pl.BlockSpec((_ROWS_PER_STEP, _V), lambda i: (i, 0)),
        ],
        out_specs=pl.BlockSpec(memory_space=pltpu.SMEM),
        out_shape=jax.ShapeDtypeStruct((1, 1), jnp.float32),
        compiler_params=pltpu.CompilerParams(
            dimension_semantics=("arbitrary",)),
    )(tgt2d, pred)


def _tc_combine_body(tc_ref, sc1_ref, m_ref, s_ref, p_ref, tgt_ref, out_ref):
    m2 = m_ref[...]
    mrow = jnp.max(m2, axis=1, keepdims=True)
    se = jnp.sum(s_ref[...] * jnp.exp(m2 - mrow), axis=1, keepdims=True)
    lse = mrow + jnp.log(se)
    p = jnp.sum(p_ref[...], axis=1, keepdims=True)
    mask = (tgt_ref[...] != _PAD).astype(jnp.float32)
    part = jnp.sum(mask * (_C + lse - _EPS * p))
    tot = tc_ref[0, 0] + jnp.sum(sc1_ref[...]) + part
    out_ref[0, 0] = tot * (1.0 / _N)


def _tc_combine(tc_scalar, sc1_parts, m2, s2, p2, tgt_sc):
    return pl.pallas_call(
        _tc_combine_body,
        in_specs=[
            pl.BlockSpec(memory_space=pltpu.SMEM),
            pl.BlockSpec((_NUM_WORKERS, _LANES), lambda: (0, 0)),
            pl.BlockSpec((_NSC, _LANES), lambda: (0, 0)),
            pl.BlockSpec((_NSC, _LANES), lambda: (0, 0)),
            pl.BlockSpec((_NSC, _LANES), lambda: (0, 0)),
            pl.BlockSpec((_NSC, 1), lambda: (0, 0)),
        ],
        out_specs=pl.BlockSpec(memory_space=pltpu.SMEM),
        out_shape=jax.ShapeDtypeStruct((1, 1), jnp.float32),
    )(tc_scalar, sc1_parts, m2, s2, p2, tgt_sc)


def kernel(pred, target):
    pred_flat = pred.reshape(-1)
    sc1_parts = _sc_gather_fn()(pred_flat, target)
    m2, s2, p2 = _sc_dense_fn()(pred_flat)
    tc_scalar = _tc_main(pred, target.reshape(_N, 1))
    loss = _tc_combine(
        tc_scalar, sc1_parts,
        m2.reshape(_NSC, _LANES), s2.reshape(_NSC, _LANES),
        p2.reshape(_NSC, _LANES), target[_NTC:].reshape(_NSC, 1))
    return loss[0, 0]


# speedup vs baseline: 7.3532x; 2.4394x over previous
import math

import jax
import jax.numpy as jnp
from jax import lax
from jax.experimental import pallas as pl
from jax.experimental.pallas import tpu as pltpu

_N = 4096
_V = 32000
_PAD = 0
_SMOOTH = 0.1
_CONF = 1.0 - _SMOOTH
_EPS = _SMOOTH / (_V - 2)
_C = _CONF * math.log(_CONF) + _SMOOTH * math.log(_EPS)
_COEF_T = _CONF - _EPS

_ROWS_PER_STEP = 128
_NUM_STEPS = _N // _ROWS_PER_STEP


def _tc_main_body(tgt_ref, x_ref, out_ref):
    i = pl.program_id(0)
    x = x_ref[...]
    m = jnp.max(x, axis=1, keepdims=True)
    se = jnp.sum(jnp.exp(x - m), axis=1, keepdims=True)
    lse = m + jnp.log(se)
    p = jnp.sum(x, axis=1, keepdims=True)
    tgt = tgt_ref[...]
    cols = lax.broadcasted_iota(jnp.int32, (_ROWS_PER_STEP, _V), 1)
    w = jnp.where(cols == tgt, -_COEF_T, 0.0) + jnp.where(cols == 0, _EPS, 0.0)
    g = jnp.sum(w * x, axis=1, keepdims=True)
    mask = (tgt != _PAD).astype(jnp.float32)
    part = jnp.sum(mask * (_C + lse - _EPS * p + g))
    prev = jnp.where(i == 0, 0.0, out_ref[0, 0])
    tot = prev + part
    out_ref[0, 0] = jnp.where(i == _NUM_STEPS - 1, tot * (1.0 / _N), tot)


def kernel(pred, target):
    loss = pl.pallas_call(
        _tc_main_body,
        grid=(_NUM_STEPS,),
        in_specs=[
            pl.BlockSpec((_ROWS_PER_STEP, 1), lambda i: (i, 0)),
            pl.BlockSpec((_ROWS_PER_STEP, _V), lambda i: (i, 0)),
        ],
        out_specs=pl.BlockSpec(memory_space=pltpu.SMEM),
        out_shape=jax.ShapeDtypeStruct((1, 1), jnp.float32),
        compiler_params=pltpu.CompilerParams(
            dimension_semantics=("arbitrary",)),
    )(target.reshape(_N, 1), pred)
    return loss[0, 0]
